# baseline (speedup 1.0000x reference)
import functools

import jax
import jax.numpy as jnp
from jax import lax
from jax.experimental import pallas as pl
from jax.experimental.pallas import tpu as pltpu
from jax.experimental.pallas import tpu_sc as plsc

N = 10000
E = 320000
D_IN = 128
D_HID = 128
D_OUT = 64

NC = 2
NS = 16
NW = NC * NS
EPW = E // NW
B = 100
CH = EPW // B
IGN = 5
IGC = CH // IGN
NP = 10240
RPT = NP // NS

_mesh = plsc.VectorSubcoreMesh(core_axis_name="c", subcore_axis_name="s")


def _fill(ref, rows, width, value):
  vec = jnp.full((16,), value, dtype=ref.dtype)

  def body(i, _):
    for j in range(width // 16):
      ref[i, pl.ds(j * 16, 16)] = vec
    return 0

  lax.fori_loop(0, rows, body, 0)


def _make_sc_spmm(D):
  cpt = 64

  @functools.partial(
      pl.kernel,
      out_type=jax.ShapeDtypeStruct((NC, NP, D), jnp.float32),
      mesh=_mesh,
      scratch_types=[
          pltpu.VMEM((IGC, B), jnp.int32),
          pltpu.VMEM((IGC, B), jnp.int32),
          pltpu.VMEM((B, D), jnp.float32),
          pltpu.VMEM((B, D), jnp.float32),
          pltpu.VMEM((cpt, D), jnp.float32),
          pltpu.VMEM_SHARED((NP, D), jnp.float32),
          pltpu.SemaphoreType.DMA,
          pltpu.SemaphoreType.DMA,
          pltpu.SemaphoreType.DMA,
          pltpu.SemaphoreType.DMA,
      ],
  )
  def spmm(table_hbm, src_hbm, dst_hbm, out_hbm, src_v, dst_v, rows0, rows1,
           stage_v, acc, sg0, sg1, ss0, ss1):
    c = lax.axis_index("c")
    s = lax.axis_index("s")
    wid = c * NS + s
    rbase = s * RPT

    pltpu.sync_copy(src_hbm.at[wid, 0], src_v)
    pltpu.sync_copy(dst_hbm.at[wid, 0], dst_v)
    pltpu.async_copy(table_hbm.at[src_v.at[0]], rows0, sg0)

    _fill(stage_v, cpt, D, 0.0)

    def zchunk(k, _):
      pltpu.sync_copy(stage_v, acc.at[pl.ds(rbase + k * cpt, cpt)])
      return 0

    lax.fori_loop(0, RPT // cpt, zchunk, 0)
    plsc.subcore_barrier()

    def group(g, _):
      @pl.when(g > 0)
      def _reload():
        pltpu.make_async_copy(rows1, acc.at[dst_v.at[IGC - 1]], ss1).wait()
        pltpu.sync_copy(src_hbm.at[wid, g], src_v)
        pltpu.sync_copy(dst_hbm.at[wid, g], dst_v)
        pltpu.async_copy(table_hbm.at[src_v.at[0]], rows0, sg0)

      def pair(t, _):
        j0 = 2 * t
        pltpu.make_async_copy(table_hbm.at[src_v.at[j0]], rows0, sg0).wait()
        pltpu.async_copy(rows0, acc.at[dst_v.at[j0]], ss0, add=True)

        @pl.when(t > 0)
        def _drain_s1():
          pltpu.make_async_copy(rows1, acc.at[dst_v.at[j0 - 1]], ss1).wait()

        pltpu.async_copy(table_hbm.at[src_v.at[j0 + 1]], rows1, sg1)
        pltpu.make_async_copy(
            table_hbm.at[src_v.at[j0 + 1]], rows1, sg1).wait()
        pltpu.async_copy(rows1, acc.at[dst_v.at[j0 + 1]], ss1, add=True)
        pltpu.make_async_copy(rows0, acc.at[dst_v.at[j0]], ss0).wait()

        @pl.when(j0 + 2 < IGC)
        def _regather0():
          pltpu.async_copy(table_hbm.at[src_v.at[j0 + 2]], rows0, sg0)

        return 0

      lax.fori_loop(0, IGC // 2, pair, 0)
      return 0

    lax.fori_loop(0, IGN, group, 0)
    pltpu.make_async_copy(rows1, acc.at[dst_v.at[IGC - 1]], ss1).wait()
    plsc.subcore_barrier()

    def ochunk(k, _):
      pltpu.sync_copy(acc.at[pl.ds(rbase + k * cpt, cpt)], stage_v)
      pltpu.sync_copy(stage_v, out_hbm.at[c, pl.ds(rbase + k * cpt, cpt)])
      return 0

    lax.fori_loop(0, RPT // cpt, ochunk, 0)

  return spmm


_sc_spmm128 = _make_sc_spmm(D_HID)


def _make_sc_deg(D):
  cpt = 64
  dq = 8

  @functools.partial(
      pl.kernel,
      out_type=jax.ShapeDtypeStruct((NC, NP, D), jnp.float32),
      mesh=_mesh,
      scratch_types=[
          pltpu.VMEM((CH, B), jnp.int32),
          pltpu.VMEM((B, D), jnp.float32),
          pltpu.VMEM((cpt, D), jnp.float32),
          pltpu.VMEM_SHARED((NP, D), jnp.float32),
          pltpu.SemaphoreType.DMA,
      ],
  )
  def deg(dst_hbm, out_hbm, dst_v, rows_v, stage_v, acc, sem):
    c = lax.axis_index("c")
    s = lax.axis_index("s")
    wid = c * NS + s
    rbase = s * RPT

    pltpu.sync_copy(dst_hbm.at[wid], dst_v)
    _fill(rows_v, B, D, 1.0)
    _fill(stage_v, cpt, D, 0.0)

    def zchunk(k, _):
      pltpu.sync_copy(stage_v, acc.at[pl.ds(rbase + k * cpt, cpt)])
      return 0

    lax.fori_loop(0, RPT // cpt, zchunk, 0)
    plsc.subcore_barrier()

    def chunk(j, _):
      @pl.when(j >= dq)
      def _retire():
        pltpu.make_async_copy(rows_v, acc.at[dst_v.at[j - dq]], sem).wait()

      pltpu.async_copy(rows_v, acc.at[dst_v.at[j]], sem, add=True)
      return 0

    lax.fori_loop(0, CH, chunk, 0)

    def drain(j, _):
      pltpu.make_async_copy(rows_v, acc.at[dst_v.at[CH - dq + j]], sem).wait()
      return 0

    lax.fori_loop(0, dq, drain, 0)
    plsc.subcore_barrier()

    def ochunk(k, _):
      pltpu.sync_copy(acc.at[pl.ds(rbase + k * cpt, cpt)], stage_v)
      pltpu.sync_copy(stage_v, out_hbm.at[c, pl.ds(rbase + k * cpt, cpt)])
      return 0

    lax.fori_loop(0, RPT // cpt, ochunk, 0)

  return deg


_sc_deg16 = _make_sc_deg(128)


R = 2000


def _tc1_body(p_ref, x_ref, w_ref, b_ref, o_ref, d_ref):
  deg = p_ref[0, :, 0:1] + p_ref[1, :, 0:1]
  d = lax.rsqrt(jnp.maximum(deg, 1.0))
  d_ref[...] = d
  sup = jnp.dot(x_ref[...], w_ref[...], preferred_element_type=jnp.float32)
  o_ref[...] = d * (sup + b_ref[...])


def _tc2_body(d_ref, acc_ref, w_ref, b_ref, o_ref):
  d = d_ref[...]
  h = jnp.maximum(d * (acc_ref[0] + acc_ref[1]), 0.0)
  sup = jnp.dot(h, w_ref[...], preferred_element_type=jnp.float32)
  o_ref[:, :D_OUT] = d * (sup + b_ref[...])
  o_ref[:, D_OUT:] = jnp.zeros((o_ref.shape[0], D_HID - D_OUT), jnp.float32)


def _tc3_body(d_ref, acc_ref, o_ref):
  d = d_ref[...]
  z = d * (acc_ref[0] + acc_ref[1])[:, :D_OUT]
  m = jnp.max(z, axis=1, keepdims=True)
  lse = m + jnp.log(jnp.sum(jnp.exp(z - m), axis=1, keepdims=True))
  o_ref[...] = z - lse


def _d_spec():
  return pl.BlockSpec((R, 1), lambda i: (i, 0))


def _tc1(deg_parts, x, W1, b1):
  return pl.pallas_call(
      _tc1_body,
      out_shape=[
          jax.ShapeDtypeStruct((N, D_HID), jnp.float32),
          jax.ShapeDtypeStruct((N, 1), jnp.float32),
      ],
      grid=(N // R,),
      in_specs=[
          pl.BlockSpec((NC, R, 16), lambda i: (0, i, 0)),
          pl.BlockSpec((R, D_IN), lambda i: (i, 0)),
          pl.BlockSpec((D_IN, D_HID), lambda i: (0, 0)),
          pl.BlockSpec((1, D_HID), lambda i: (0, 0)),
      ],
      out_specs=[
          pl.BlockSpec((R, D_HID), lambda i: (i, 0)),
          pl.BlockSpec((R, 1), lambda i: (i, 0)),
      ],
  )(deg_parts, x, W1, b1.reshape(1, D_HID))


def _tc2(d_col, acc1, W2, b2):
  return pl.pallas_call(
      _tc2_body,
      out_shape=jax.ShapeDtypeStruct((N, D_HID), jnp.float32),
      grid=(N // R,),
      in_specs=[
          _d_spec(),
          pl.BlockSpec((NC, R, D_HID), lambda i: (0, i, 0)),
          pl.BlockSpec((D_HID, D_OUT), lambda i: (0, 0)),
          pl.BlockSpec((1, D_OUT), lambda i: (0, 0)),
      ],
      out_specs=pl.BlockSpec((R, D_HID), lambda i: (i, 0)),
  )(d_col, acc1, W2, b2.reshape(1, D_OUT))


def _tc3(d_col, acc2):
  return pl.pallas_call(
      _tc3_body,
      out_shape=jax.ShapeDtypeStruct((N, D_OUT), jnp.float32),
      grid=(N // R,),
      in_specs=[
          _d_spec(),
          pl.BlockSpec((NC, R, D_HID), lambda i: (0, i, 0)),
      ],
      out_specs=pl.BlockSpec((R, D_OUT), lambda i: (i, 0)),
  )(d_col, acc2)


def kernel(x, preprocessed, W1, b1, W2, b2):
  src = preprocessed[0].reshape(NW, IGN, IGC, B)
  dst = preprocessed[1].reshape(NW, IGN, IGC, B)
  deg_parts = _sc_deg16(preprocessed[1].reshape(NW, CH, B))[:, :, :16]
  s1, d_col = _tc1(deg_parts, x, W1, b1)
  acc1 = _sc_spmm128(s1, src, dst)
  s2 = _tc2(d_col, acc1, W2, b2)
  acc2 = _sc_spmm128(s2, src, dst)
  return _tc3(d_col, acc2)


# speedup vs baseline: 1.1428x; 1.1428x over previous
import functools

import jax
import jax.numpy as jnp
from jax import lax
from jax.experimental import pallas as pl
from jax.experimental.pallas import tpu as pltpu
from jax.experimental.pallas import tpu_sc as plsc

N = 10000
E = 320000
D_IN = 128
D_HID = 128
D_OUT = 64

NC = 2
NS = 16
NW = NC * NS
EPW = E // NW
B = 100
CH = EPW // B
IGN = 5
IGC = CH // IGN
NP = 10240
RPT = NP // NS

_mesh = plsc.VectorSubcoreMesh(core_axis_name="c", subcore_axis_name="s")


def _fill(ref, rows, width, value):
  vec = jnp.full((16,), value, dtype=ref.dtype)

  def body(i, _):
    for j in range(width // 16):
      ref[i, pl.ds(j * 16, 16)] = vec
    return 0

  lax.fori_loop(0, rows, body, 0)


def _make_sc_spmm(D):
  cpt = 64

  @functools.partial(
      pl.kernel,
      out_type=jax.ShapeDtypeStruct((NC, NP, D), jnp.float32),
      mesh=_mesh,
      scratch_types=[
          pltpu.VMEM((IGC, B), jnp.int32),
          pltpu.VMEM((IGC, B), jnp.int32),
          pltpu.VMEM((B, D), jnp.float32),
          pltpu.VMEM((B, D), jnp.float32),
          pltpu.VMEM((cpt, D), jnp.float32),
          pltpu.VMEM_SHARED((NP, D), jnp.float32),
          pltpu.SemaphoreType.DMA,
          pltpu.SemaphoreType.DMA,
          pltpu.SemaphoreType.DMA,
          pltpu.SemaphoreType.DMA,
      ],
  )
  def spmm(table_hbm, src_hbm, dst_hbm, out_hbm, src_v, dst_v, rows0, rows1,
           stage_v, acc, sg0, sg1, ss0, ss1):
    c = lax.axis_index("c")
    s = lax.axis_index("s")
    wid = c * NS + s
    rbase = s * RPT

    pltpu.sync_copy(src_hbm.at[wid, 0], src_v)
    pltpu.sync_copy(dst_hbm.at[wid, 0], dst_v)
    pltpu.async_copy(table_hbm.at[src_v.at[0]], rows0, sg0)

    _fill(stage_v, cpt, D, 0.0)

    def zchunk(k, _):
      pltpu.sync_copy(stage_v, acc.at[pl.ds(rbase + k * cpt, cpt)])
      return 0

    lax.fori_loop(0, RPT // cpt, zchunk, 0)
    plsc.subcore_barrier()

    def group(g, _):
      @pl.when(g > 0)
      def _reload():
        pltpu.make_async_copy(rows1, acc.at[dst_v.at[IGC - 1]], ss1).wait()
        pltpu.sync_copy(src_hbm.at[wid, g], src_v)
        pltpu.sync_copy(dst_hbm.at[wid, g], dst_v)
        pltpu.async_copy(table_hbm.at[src_v.at[0]], rows0, sg0)

      def pair(t, _):
        j0 = 2 * t

        @pl.when(t > 0)
        def _drain_s1():
          pltpu.make_async_copy(rows1, acc.at[dst_v.at[j0 - 1]], ss1).wait()

        pltpu.async_copy(table_hbm.at[src_v.at[j0 + 1]], rows1, sg1)
        pltpu.make_async_copy(table_hbm.at[src_v.at[j0]], rows0, sg0).wait()
        pltpu.async_copy(rows0, acc.at[dst_v.at[j0]], ss0, add=True)
        pltpu.make_async_copy(
            table_hbm.at[src_v.at[j0 + 1]], rows1, sg1).wait()
        pltpu.async_copy(rows1, acc.at[dst_v.at[j0 + 1]], ss1, add=True)
        pltpu.make_async_copy(rows0, acc.at[dst_v.at[j0]], ss0).wait()

        @pl.when(j0 + 2 < IGC)
        def _regather0():
          pltpu.async_copy(table_hbm.at[src_v.at[j0 + 2]], rows0, sg0)

        return 0

      lax.fori_loop(0, IGC // 2, pair, 0)
      return 0

    lax.fori_loop(0, IGN, group, 0)
    pltpu.make_async_copy(rows1, acc.at[dst_v.at[IGC - 1]], ss1).wait()
    plsc.subcore_barrier()

    def ochunk(k, _):
      pltpu.sync_copy(acc.at[pl.ds(rbase + k * cpt, cpt)], stage_v)
      pltpu.sync_copy(stage_v, out_hbm.at[c, pl.ds(rbase + k * cpt, cpt)])
      return 0

    lax.fori_loop(0, RPT // cpt, ochunk, 0)

  return spmm


_sc_spmm128 = _make_sc_spmm(D_HID)


def _make_sc_deg(D):
  cpt = 64
  dq = 8

  @functools.partial(
      pl.kernel,
      out_type=jax.ShapeDtypeStruct((NC, NP, D), jnp.float32),
      mesh=_mesh,
      scratch_types=[
          pltpu.VMEM((CH, B), jnp.int32),
          pltpu.VMEM((B, D), jnp.float32),
          pltpu.VMEM((cpt, D), jnp.float32),
          pltpu.VMEM_SHARED((NP, D), jnp.float32),
          pltpu.SemaphoreType.DMA,
      ],
  )
  def deg(dst_hbm, out_hbm, dst_v, rows_v, stage_v, acc, sem):
    c = lax.axis_index("c")
    s = lax.axis_index("s")
    wid = c * NS + s
    rbase = s * RPT

    pltpu.sync_copy(dst_hbm.at[wid], dst_v)
    _fill(rows_v, B, D, 1.0)
    _fill(stage_v, cpt, D, 0.0)

    def zchunk(k, _):
      pltpu.sync_copy(stage_v, acc.at[pl.ds(rbase + k * cpt, cpt)])
      return 0

    lax.fori_loop(0, RPT // cpt, zchunk, 0)
    plsc.subcore_barrier()

    def chunk(j, _):
      @pl.when(j >= dq)
      def _retire():
        pltpu.make_async_copy(rows_v, acc.at[dst_v.at[j - dq]], sem).wait()

      pltpu.async_copy(rows_v, acc.at[dst_v.at[j]], sem, add=True)
      return 0

    lax.fori_loop(0, CH, chunk, 0)

    def drain(j, _):
      pltpu.make_async_copy(rows_v, acc.at[dst_v.at[CH - dq + j]], sem).wait()
      return 0

    lax.fori_loop(0, dq, drain, 0)
    plsc.subcore_barrier()

    def ochunk(k, _):
      pltpu.sync_copy(acc.at[pl.ds(rbase + k * cpt, cpt)], stage_v)
      pltpu.sync_copy(stage_v, out_hbm.at[c, pl.ds(rbase + k * cpt, cpt)])
      return 0

    lax.fori_loop(0, RPT // cpt, ochunk, 0)

  return deg


_sc_deg16 = _make_sc_deg(128)


R = 2000


def _tc1_body(p_ref, x_ref, w_ref, b_ref, o_ref, d_ref):
  deg = p_ref[0, :, 0:1] + p_ref[1, :, 0:1]
  d = lax.rsqrt(jnp.maximum(deg, 1.0))
  d_ref[...] = d
  sup = jnp.dot(x_ref[...], w_ref[...], preferred_element_type=jnp.float32)
  o_ref[...] = d * (sup + b_ref[...])


def _tc2_body(d_ref, acc_ref, w_ref, b_ref, o_ref):
  d = d_ref[...]
  h = jnp.maximum(d * (acc_ref[0] + acc_ref[1]), 0.0)
  sup = jnp.dot(h, w_ref[...], preferred_element_type=jnp.float32)
  o_ref[:, :D_OUT] = d * (sup + b_ref[...])
  o_ref[:, D_OUT:] = jnp.zeros((o_ref.shape[0], D_HID - D_OUT), jnp.float32)


def _tc3_body(d_ref, acc_ref, o_ref):
  d = d_ref[...]
  z = d * (acc_ref[0] + acc_ref[1])[:, :D_OUT]
  m = jnp.max(z, axis=1, keepdims=True)
  lse = m + jnp.log(jnp.sum(jnp.exp(z - m), axis=1, keepdims=True))
  o_ref[...] = z - lse


def _d_spec():
  return pl.BlockSpec((R, 1), lambda i: (i, 0))


def _tc1(deg_parts, x, W1, b1):
  return pl.pallas_call(
      _tc1_body,
      out_shape=[
          jax.ShapeDtypeStruct((N, D_HID), jnp.float32),
          jax.ShapeDtypeStruct((N, 1), jnp.float32),
      ],
      grid=(N // R,),
      in_specs=[
          pl.BlockSpec((NC, R, 16), lambda i: (0, i, 0)),
          pl.BlockSpec((R, D_IN), lambda i: (i, 0)),
          pl.BlockSpec((D_IN, D_HID), lambda i: (0, 0)),
          pl.BlockSpec((1, D_HID), lambda i: (0, 0)),
      ],
      out_specs=[
          pl.BlockSpec((R, D_HID), lambda i: (i, 0)),
          pl.BlockSpec((R, 1), lambda i: (i, 0)),
      ],
  )(deg_parts, x, W1, b1.reshape(1, D_HID))


def _tc2(d_col, acc1, W2, b2):
  return pl.pallas_call(
      _tc2_body,
      out_shape=jax.ShapeDtypeStruct((N, D_HID), jnp.float32),
      grid=(N // R,),
      in_specs=[
          _d_spec(),
          pl.BlockSpec((NC, R, D_HID), lambda i: (0, i, 0)),
          pl.BlockSpec((D_HID, D_OUT), lambda i: (0, 0)),
          pl.BlockSpec((1, D_OUT), lambda i: (0, 0)),
      ],
      out_specs=pl.BlockSpec((R, D_HID), lambda i: (i, 0)),
  )(d_col, acc1, W2, b2.reshape(1, D_OUT))


def _tc3(d_col, acc2):
  return pl.pallas_call(
      _tc3_body,
      out_shape=jax.ShapeDtypeStruct((N, D_OUT), jnp.float32),
      grid=(N // R,),
      in_specs=[
          _d_spec(),
          pl.BlockSpec((NC, R, D_HID), lambda i: (0, i, 0)),
      ],
      out_specs=pl.BlockSpec((R, D_OUT), lambda i: (i, 0)),
  )(d_col, acc2)


def kernel(x, preprocessed, W1, b1, W2, b2):
  src = preprocessed[0].reshape(NW, IGN, IGC, B)
  dst = preprocessed[1].reshape(NW, IGN, IGC, B)
  deg_parts = _sc_deg16(preprocessed[1].reshape(NW, CH, B))[:, :, :16]
  s1, d_col = _tc1(deg_parts, x, W1, b1)
  acc1 = _sc_spmm128(s1, src, dst)
  s2 = _tc2(d_col, acc1, W2, b2)
  acc2 = _sc_spmm128(s2, src, dst)
  return _tc3(d_col, acc2)
